# Initial kernel scaffold; baseline (speedup 1.0000x reference)
import jax
import jax.numpy as jnp
from jax.experimental import pallas as pl


def kernel(x, edge_index, W1, b1, W2, b2):
    raise NotImplementedError("write your pallas kernel here")


# speedup vs baseline: 16.3220x; 16.3220x over previous
import functools

import jax
import jax.numpy as jnp
from jax import lax
from jax.experimental import pallas as pl
from jax.experimental.pallas import tpu as pltpu
from jax.experimental.pallas import tpu_sc as plsc

N = 10000
D = 128
E = 320000
N_ACT = 8

NPAD = 10240
CH = 128
NTILES = 32
ECHUNKS = 2560
EPAD = ECHUNKS * CH
CPT = ECHUNKS // NTILES
HCH = CPT // 2
PAD_SRC = N
PAD_DST = N + 16
DEGW = 16
BM = 1280

_SC_MESH = dict(core_axis_name="c", subcore_axis_name="s")


@functools.partial(
    pl.kernel,
    out_type=jax.ShapeDtypeStruct((2, NPAD, DEGW), jnp.float32),
    mesh=plsc.VectorSubcoreMesh(**_SC_MESH),
    scratch_types=[
        pltpu.VMEM((CPT, CH), jnp.int32),
        pltpu.VMEM((CH, DEGW), jnp.float32),
        pltpu.VMEM((64, DEGW), jnp.float32),
        pltpu.VMEM_SHARED((NPAD, DEGW), jnp.float32),
    ],
)
def _deg_kernel(dst_hbm, out_hbm, didx, ones_v, zbuf, deg_sh):
    cid = lax.axis_index("c")
    sid = lax.axis_index("s")
    wid = cid * 16 + sid
    io = lax.iota(jnp.int32, 16)
    one_row = jnp.where(io == 0, 1.0, 0.0).astype(jnp.float32)
    zrow = jnp.zeros((16,), jnp.float32)

    def _fill(i, _):
        ones_v[i, :] = one_row
        zbuf[i % 64, :] = zrow
        return 0

    lax.fori_loop(0, CH, _fill, 0)

    for k in range(10):
        pltpu.sync_copy(zbuf, deg_sh.at[pl.ds(sid * 640 + k * 64, 64)])
    plsc.subcore_barrier()

    pltpu.sync_copy(dst_hbm.at[pl.ds(wid * CPT, CPT)], didx)

    def _body(j, _):
        pltpu.sync_copy(ones_v, deg_sh.at[didx.at[j]], add=True)
        return 0

    lax.fori_loop(0, CPT, _body, 0)
    plsc.subcore_barrier()
    pltpu.sync_copy(deg_sh.at[pl.ds(sid * 640, 640)],
                    out_hbm.at[cid, pl.ds(sid * 640, 640)])


@functools.partial(
    pl.kernel,
    out_type=jax.ShapeDtypeStruct((2, NPAD, D), jnp.float32),
    mesh=plsc.VectorSubcoreMesh(**_SC_MESH),
    scratch_types=[
        pltpu.VMEM((HCH, CH), jnp.int32),
        pltpu.VMEM((HCH, CH), jnp.int32),
        pltpu.VMEM((CH, D), jnp.float32),
        pltpu.VMEM((CH, D), jnp.float32),
        pltpu.SemaphoreType.DMA,
        pltpu.SemaphoreType.DMA,
        pltpu.VMEM_SHARED((NPAD, D), jnp.float32),
    ],
)
def _agg_kernel(src_hbm, dst_hbm, y_hbm, out_hbm,
                sidx, didx, buf0, buf1, sem0, sem1, acc_sh):
    cid = lax.axis_index("c")
    sid = lax.axis_index("s")
    wid = cid * 16 + sid
    zrow = jnp.zeros((16,), jnp.float32)

    def _zero(i, _):
        for j in range(8):
            buf0[i, pl.ds(j * 16, 16)] = zrow
        return 0

    lax.fori_loop(0, CH, _zero, 0)
    for k in range(5):
        pltpu.sync_copy(buf0, acc_sh.at[pl.ds(sid * 640 + k * CH, CH)])
    plsc.subcore_barrier()

    def _start(j, buf, sem):
        pltpu.async_copy(y_hbm.at[sidx.at[j]], buf, sem)

    def _wait(j, buf, sem):
        pltpu.make_async_copy(y_hbm.at[sidx.at[j]], buf, sem).wait()

    def _scatter(j, buf):
        pltpu.sync_copy(buf, acc_sh.at[didx.at[j]], add=True)

    for h in range(2):
        base = wid * CPT + h * HCH
        pltpu.sync_copy(src_hbm.at[pl.ds(base, HCH)], sidx)
        pltpu.sync_copy(dst_hbm.at[pl.ds(base, HCH)], didx)
        _start(0, buf0, sem0)

        def _body(jj, _):
            j0 = jj * 2
            j1 = j0 + 1
            _start(j1, buf1, sem1)
            _wait(j0, buf0, sem0)
            _scatter(j0, buf0)

            @pl.when(jj < HCH // 2 - 1)
            def _():
                _start(j0 + 2, buf0, sem0)

            _wait(j1, buf1, sem1)
            _scatter(j1, buf1)
            return 0

        lax.fori_loop(0, HCH // 2, _body, 0)

    plsc.subcore_barrier()
    pltpu.sync_copy(acc_sh.at[pl.ds(sid * 640, 640)],
                    out_hbm.at[cid, pl.ds(sid * 640, 640)])


def _y_body(x_ref, w_ref, dg_ref, y_ref):
    dg = dg_ref[...]
    deg = dg[0, :, :1] + dg[1, :, :1] + 1.0
    dis = lax.rsqrt(deg)
    y_ref[...] = jnp.dot(x_ref[...], w_ref[...],
                         preferred_element_type=jnp.float32) * dis


def _y_call(xp, W1, degp):
    return pl.pallas_call(
        _y_body,
        grid=(NPAD // BM,),
        in_specs=[
            pl.BlockSpec((BM, D), lambda i: (i, 0)),
            pl.BlockSpec((D, D), lambda i: (0, 0)),
            pl.BlockSpec((2, BM, DEGW), lambda i: (0, i, 0)),
        ],
        out_specs=pl.BlockSpec((BM, D), lambda i: (i, 0)),
        out_shape=jax.ShapeDtypeStruct((NPAD, D), jnp.float32),
    )(xp, W1, degp)


def _fin_body(agg_ref, y_ref, dg_ref, b1_ref, w2_ref, b2_ref, o_ref):
    dg = dg_ref[...]
    deg = dg[0, :, :1] + dg[1, :, :1] + 1.0
    dis = lax.rsqrt(deg)
    a = agg_ref[...]
    s = a[0] + a[1] + y_ref[...]
    h = jnp.tanh(dis * s + b1_ref[...])
    o_ref[...] = jnp.dot(h, w2_ref[...],
                         preferred_element_type=jnp.float32) + b2_ref[...]


def _fin_call(agg, y, degp, b1, W2, b2):
    return pl.pallas_call(
        _fin_body,
        grid=(NPAD // BM,),
        in_specs=[
            pl.BlockSpec((2, BM, D), lambda i: (0, i, 0)),
            pl.BlockSpec((BM, D), lambda i: (i, 0)),
            pl.BlockSpec((2, BM, DEGW), lambda i: (0, i, 0)),
            pl.BlockSpec((1, D), lambda i: (0, 0)),
            pl.BlockSpec((D, N_ACT), lambda i: (0, 0)),
            pl.BlockSpec((1, N_ACT), lambda i: (0, 0)),
        ],
        out_specs=pl.BlockSpec((BM, N_ACT), lambda i: (i, 0)),
        out_shape=jax.ShapeDtypeStruct((NPAD, N_ACT), jnp.float32),
    )(agg, y, degp, b1, W2, b2)


def kernel(x, edge_index, W1, b1, W2, b2):
    ei = edge_index.astype(jnp.int32)
    pad = EPAD - E
    srcp = jnp.concatenate(
        [ei[0], jnp.full((pad,), PAD_SRC, jnp.int32)]).reshape(ECHUNKS, CH)
    dstp = jnp.concatenate(
        [ei[1], jnp.full((pad,), PAD_DST, jnp.int32)]).reshape(ECHUNKS, CH)
    xp = jnp.pad(x, ((0, NPAD - N), (0, 0)))

    degp = _deg_kernel(dstp)
    y = _y_call(xp, W1, degp)
    agg = _agg_kernel(srcp, dstp, y)
    out = _fin_call(agg, y, degp, b1.reshape(1, D), W2, b2.reshape(1, N_ACT))
    return out[:N]
